# baseline (speedup 1.0000x reference)
import functools

import jax
import jax.numpy as jnp
from jax import lax
from jax.experimental import pallas as pl
from jax.experimental.pallas import tpu as pltpu
from jax.experimental.pallas import tpu_sc as plsc

N = 100000
D = 128
N_TYPES = 4
L = 16
NC, NS = 2, 16
NW = NC * NS


BLOCK = 32768


def _tc_body(w_ref, d_ref, s0_ref, s1_ref, s2_ref, s3_ref):
    st = lax.dot_general(
        w_ref[...], d_ref[...],
        dimension_numbers=(((1,), (1,)), ((), ())),
        preferred_element_type=jnp.float32)
    s0_ref[...] = st[0, :]
    s1_ref[...] = st[1, :]
    s2_ref[...] = st[2, :]
    s3_ref[...] = st[3, :]


_tc_matmul = pl.pallas_call(
    _tc_body,
    grid=(pl.cdiv(N, BLOCK),),
    in_specs=[
        pl.BlockSpec((N_TYPES, D), lambda i: (0, 0)),
        pl.BlockSpec((BLOCK, D), lambda i: (i, 0)),
    ],
    out_specs=[pl.BlockSpec((BLOCK,), lambda i: (i,))] * N_TYPES,
    out_shape=[jax.ShapeDtypeStruct((N,), jnp.float32)] * N_TYPES,
)


CHUNK = 2000
NCHUNKS = N // CHUNK
ITERS = (NCHUNKS + NW - 1) // NW
PAIRS = ITERS // 2


def _sc_body(st0_hbm, st1_hbm, st2_hbm, st3_hbm, elems_hbm, b_hbm, out_hbm,
             sbuf0, sbuf1, ebuf0, ebuf1, obuf0, obuf1, bbuf,
             isem0, isem1, osem0, osem1):
    wid = lax.axis_index("s") * NC + lax.axis_index("c")
    sbuf = (sbuf0, sbuf1)
    ebuf = (ebuf0, ebuf1)
    obuf = (obuf0, obuf1)
    isem = (isem0, isem1)
    osem = (osem0, osem1)
    st_hbm = (st0_hbm, st1_hbm, st2_hbm, st3_hbm)
    pltpu.sync_copy(b_hbm, bbuf)

    def start_in(b, c):
        for t in range(N_TYPES):
            pltpu.async_copy(st_hbm[t].at[pl.ds(c * CHUNK, CHUNK)],
                             sbuf[b].at[pl.ds(t * CHUNK, CHUNK)], isem[b])
        pltpu.async_copy(elems_hbm.at[pl.ds(c * CHUNK, CHUNK)], ebuf[b],
                         isem[b])

    def wait_in(b):
        for t in range(N_TYPES):
            pltpu.make_async_copy(st_hbm[0].at[pl.ds(0, CHUNK)],
                                  sbuf[b].at[pl.ds(0, CHUNK)], isem[b]).wait()
        pltpu.make_async_copy(elems_hbm.at[pl.ds(0, CHUNK)], ebuf[b],
                              isem[b]).wait()

    def start_out(b, c):
        pltpu.async_copy(obuf[b], out_hbm.at[pl.ds(c * CHUNK, CHUNK)],
                         osem[b])

    def wait_out(b):
        pltpu.make_async_copy(obuf[b], out_hbm.at[pl.ds(0, CHUNK)],
                              osem[b]).wait()

    def compute(b):
        sb, eb, ob = sbuf[b], ebuf[b], obuf[b]
        bias = bbuf[pl.ds(0, L)]

        @plsc.parallel_loop(0, CHUNK // L, unroll=4)
        def group_body(g):
            ev = eb[pl.ds(g * L, L)]
            c0 = sb[pl.ds(0 * CHUNK + g * L, L)]
            c1 = sb[pl.ds(1 * CHUNK + g * L, L)]
            c2 = sb[pl.ds(2 * CHUNK + g * L, L)]
            c3 = sb[pl.ds(3 * CHUNK + g * L, L)]
            sel = jnp.where(ev == 0, c0,
                            jnp.where(ev == 1, c1,
                                      jnp.where(ev == 2, c2, c3)))
            ob[pl.ds(g * L, L)] = sel + bias

    start_in(0, wid)

    @pl.when(NW + wid < NCHUNKS)
    def _():
        start_in(1, NW + wid)

    def pair_body(p, _):
        for b in (0, 1):
            i = 2 * p + b
            c = i * NW + wid
            cn2 = c + 2 * NW

            @pl.when(c < NCHUNKS)
            def _():
                wait_in(b)

                @pl.when(p >= 1)
                def _():
                    wait_out(b)

                compute(b)
                start_out(b, c)

                @pl.when(cn2 < NCHUNKS)
                def _():
                    start_in(b, cn2)

        return 0

    lax.fori_loop(0, PAIRS, pair_body, 0)

    wait_out(0)

    @pl.when((2 * PAIRS - 1) * NW + wid < NCHUNKS)
    def _():
        wait_out(1)


_mesh = plsc.VectorSubcoreMesh(core_axis_name="c", subcore_axis_name="s")

_sc_select = functools.partial(
    pl.kernel,
    mesh=_mesh,
    compiler_params=pltpu.CompilerParams(needs_layout_passes=False),
    out_type=jax.ShapeDtypeStruct((N,), jnp.float32),
    scratch_types=[
        pltpu.VMEM((CHUNK * N_TYPES,), jnp.float32),
        pltpu.VMEM((CHUNK * N_TYPES,), jnp.float32),
        pltpu.VMEM((CHUNK,), jnp.int32),
        pltpu.VMEM((CHUNK,), jnp.int32),
        pltpu.VMEM((CHUNK,), jnp.float32),
        pltpu.VMEM((CHUNK,), jnp.float32),
        pltpu.VMEM((L,), jnp.float32),
        pltpu.SemaphoreType.DMA,
        pltpu.SemaphoreType.DMA,
        pltpu.SemaphoreType.DMA,
        pltpu.SemaphoreType.DMA,
    ],
)(_sc_body)


def kernel(descriptors, elems, W, b):
    wt = W.reshape(N_TYPES, D)
    s0, s1, s2, s3 = _tc_matmul(wt, descriptors)
    bb = jnp.broadcast_to(b, (L,))
    dots = _sc_select(s0, s1, s2, s3, elems.astype(jnp.int32), bb)
    return dots.reshape(N, 1)


# speedup vs baseline: 1.0589x; 1.0589x over previous
import functools

import jax
import jax.numpy as jnp
from jax import lax
from jax.experimental import pallas as pl
from jax.experimental.pallas import tpu as pltpu
from jax.experimental.pallas import tpu_sc as plsc

N = 100000
D = 128
N_TYPES = 4
L = 16
NC, NS = 2, 16
NW = NC * NS


BLOCK = 16384


def _tc_body(w_ref, d_ref, s0_ref, s1_ref, s2_ref, s3_ref):
    st = lax.dot_general(
        w_ref[...], d_ref[...],
        dimension_numbers=(((1,), (1,)), ((), ())),
        preferred_element_type=jnp.float32)
    s0_ref[...] = st[0, :]
    s1_ref[...] = st[1, :]
    s2_ref[...] = st[2, :]
    s3_ref[...] = st[3, :]


_tc_matmul = pl.pallas_call(
    _tc_body,
    grid=(pl.cdiv(N, BLOCK),),
    in_specs=[
        pl.BlockSpec((N_TYPES, D), lambda i: (0, 0)),
        pl.BlockSpec((BLOCK, D), lambda i: (i, 0)),
    ],
    out_specs=[pl.BlockSpec((BLOCK,), lambda i: (i,))] * N_TYPES,
    out_shape=[jax.ShapeDtypeStruct((N,), jnp.float32)] * N_TYPES,
)


CHUNK = 2000
NCHUNKS = N // CHUNK
ITERS = (NCHUNKS + NW - 1) // NW
PAIRS = ITERS // 2


def _sc_body(st0_hbm, st1_hbm, st2_hbm, st3_hbm, elems_hbm, b_hbm, out_hbm,
             sbuf0, sbuf1, ebuf0, ebuf1, obuf0, obuf1, bbuf,
             isem0, isem1, osem0, osem1):
    wid = lax.axis_index("s") * NC + lax.axis_index("c")
    sbuf = (sbuf0, sbuf1)
    ebuf = (ebuf0, ebuf1)
    obuf = (obuf0, obuf1)
    isem = (isem0, isem1)
    osem = (osem0, osem1)
    st_hbm = (st0_hbm, st1_hbm, st2_hbm, st3_hbm)
    pltpu.sync_copy(b_hbm, bbuf)

    def start_in(b, c):
        for t in range(N_TYPES):
            pltpu.async_copy(st_hbm[t].at[pl.ds(c * CHUNK, CHUNK)],
                             sbuf[b].at[pl.ds(t * CHUNK, CHUNK)], isem[b])
        pltpu.async_copy(elems_hbm.at[pl.ds(c * CHUNK, CHUNK)], ebuf[b],
                         isem[b])

    def wait_in(b):
        for t in range(N_TYPES):
            pltpu.make_async_copy(st_hbm[0].at[pl.ds(0, CHUNK)],
                                  sbuf[b].at[pl.ds(0, CHUNK)], isem[b]).wait()
        pltpu.make_async_copy(elems_hbm.at[pl.ds(0, CHUNK)], ebuf[b],
                              isem[b]).wait()

    def start_out(b, c):
        pltpu.async_copy(obuf[b], out_hbm.at[pl.ds(c * CHUNK, CHUNK)],
                         osem[b])

    def wait_out(b):
        pltpu.make_async_copy(obuf[b], out_hbm.at[pl.ds(0, CHUNK)],
                              osem[b]).wait()

    def compute(b):
        sb, eb, ob = sbuf[b], ebuf[b], obuf[b]
        bias = bbuf[pl.ds(0, L)]

        @plsc.parallel_loop(0, CHUNK // L, unroll=4)
        def group_body(g):
            ev = eb[pl.ds(g * L, L)]
            c0 = sb[pl.ds(0 * CHUNK + g * L, L)]
            c1 = sb[pl.ds(1 * CHUNK + g * L, L)]
            c2 = sb[pl.ds(2 * CHUNK + g * L, L)]
            c3 = sb[pl.ds(3 * CHUNK + g * L, L)]
            sel = jnp.where(ev == 0, c0,
                            jnp.where(ev == 1, c1,
                                      jnp.where(ev == 2, c2, c3)))
            ob[pl.ds(g * L, L)] = sel + bias

    start_in(0, wid)

    @pl.when(NW + wid < NCHUNKS)
    def _():
        start_in(1, NW + wid)

    def pair_body(p, _):
        for b in (0, 1):
            i = 2 * p + b
            c = i * NW + wid
            cn2 = c + 2 * NW

            @pl.when(c < NCHUNKS)
            def _():
                wait_in(b)

                @pl.when(p >= 1)
                def _():
                    wait_out(b)

                compute(b)
                start_out(b, c)

                @pl.when(cn2 < NCHUNKS)
                def _():
                    start_in(b, cn2)

        return 0

    lax.fori_loop(0, PAIRS, pair_body, 0)

    wait_out(0)

    @pl.when((2 * PAIRS - 1) * NW + wid < NCHUNKS)
    def _():
        wait_out(1)


_mesh = plsc.VectorSubcoreMesh(core_axis_name="c", subcore_axis_name="s")

_sc_select = functools.partial(
    pl.kernel,
    mesh=_mesh,
    compiler_params=pltpu.CompilerParams(needs_layout_passes=False),
    out_type=jax.ShapeDtypeStruct((N,), jnp.float32),
    scratch_types=[
        pltpu.VMEM((CHUNK * N_TYPES,), jnp.float32),
        pltpu.VMEM((CHUNK * N_TYPES,), jnp.float32),
        pltpu.VMEM((CHUNK,), jnp.int32),
        pltpu.VMEM((CHUNK,), jnp.int32),
        pltpu.VMEM((CHUNK,), jnp.float32),
        pltpu.VMEM((CHUNK,), jnp.float32),
        pltpu.VMEM((L,), jnp.float32),
        pltpu.SemaphoreType.DMA,
        pltpu.SemaphoreType.DMA,
        pltpu.SemaphoreType.DMA,
        pltpu.SemaphoreType.DMA,
    ],
)(_sc_body)


def kernel(descriptors, elems, W, b):
    wt = W.reshape(N_TYPES, D)
    s0, s1, s2, s3 = _tc_matmul(wt, descriptors)
    bb = jnp.broadcast_to(b, (L,))
    dots = _sc_select(s0, s1, s2, s3, elems.astype(jnp.int32), bb)
    return dots.reshape(N, 1)
